# baseline (speedup 1.0000x reference)
import functools

import jax
import jax.numpy as jnp
from jax import lax
from jax.experimental import pallas as pl
from jax.experimental.pallas import tpu as pltpu
from jax.experimental.pallas import tpu_sc as plsc

_NUM_CORES = 2
_NUM_SUBCORES = 16
_NUM_WORKERS = _NUM_CORES * _NUM_SUBCORES
_PAD = 128
_LANES = 16
_NBUF = 4
_WBUF = 2


def _make_lookup(num_rows: int, dim: int, s0: int, s1: int):
  assert s0 % _NUM_WORKERS == 0
  bpw = s0 // _NUM_WORKERS
  assert bpw == 128
  kt = dim // 8
  jt = s0 // 128

  mesh = plsc.VectorSubcoreMesh(
      core_axis_name="c", subcore_axis_name="s", num_cores=_NUM_CORES)

  @functools.partial(
      pl.kernel,
      mesh=mesh,
      compiler_params=pltpu.CompilerParams(
          use_tc_tiling_on_sc=False, needs_layout_passes=False),
      out_type=jax.ShapeDtypeStruct((s1, kt, jt, 8, 128), jnp.float32),
      scratch_types=[
          pltpu.VMEM((s1, bpw), jnp.int32),
          pltpu.VMEM((_NBUF, bpw, dim), jnp.float32),
          pltpu.VMEM((_WBUF, kt, 8, 128), jnp.float32),
          pltpu.SemaphoreType.DMA,
          pltpu.SemaphoreType.DMA,
      ],
  )
  def lookup(table_hbm, idxt_hbm, out_hbm, idx_v, rows_v, stage_v, gsem,
             wsem):
    wid = lax.axis_index("s") * _NUM_CORES + lax.axis_index("c")
    pltpu.sync_copy(idxt_hbm.at[:, pl.ds(wid * bpw, bpw)], idx_v)

    lane = lax.iota(jnp.int32, _LANES)
    row_sel = [lane + (m * _LANES) for m in range(bpw // _LANES)]
    col_sel = [
        jnp.full((_LANES,), d, jnp.int32) for d in range(dim)
    ]

    def start_gather(t, b):
      return pltpu.async_copy(
          table_hbm.at[idx_v.at[t]], rows_v.at[b], gsem)

    def wait_gather(b):
      pltpu.make_async_copy(
          table_hbm.at[idx_v.at[0]], rows_v.at[b], gsem).wait()

    def wait_write(b):
      pltpu.make_async_copy(
          stage_v.at[b], out_hbm.at[0].at[:, wid], wsem).wait()

    for b in range(_NBUF):
      start_gather(b, b)

    n_steps = s1 // _NBUF

    @pl.loop(0, n_steps)
    def _step(g):
      for b in range(_NBUF):
        t = g * _NBUF + b
        wait_gather(b)
        w = b % _WBUF
        @pl.when(t >= _WBUF)
        def _():
          wait_write(w)
        src = rows_v.at[b]
        dst = stage_v.at[w]
        for k in range(kt):
          for dr in range(8):
            d = k * 8 + dr
            for m in range(bpw // _LANES):
              v = plsc.load_gather(src, [row_sel[m], col_sel[d]])
              dst[k, dr, pl.ds(m * _LANES, _LANES)] = v
        pltpu.async_copy(dst, out_hbm.at[t].at[:, wid], wsem)
        @pl.when(t + _NBUF < s1)
        def _():
          start_gather(t + _NBUF, b)

    for w in range(_WBUF):
      wait_write(w)

  return lookup


def kernel(token_ids, embedding_matrix):
  s0, s1 = token_ids.shape
  num_rows, dim = embedding_matrix.shape
  idxt = (token_ids.astype(jnp.int32) * 2).T
  proj = jnp.eye(dim, _PAD, dtype=jnp.float32)
  tbl = jax.lax.dot(embedding_matrix, proj,
                    precision=jax.lax.Precision.HIGHEST)
  tbl2 = tbl.reshape(num_rows * 2, dim)
  lookup = _make_lookup(num_rows, dim, s0, s1)
  img = lookup(tbl2, idxt)
  return jnp.transpose(img, (2, 4, 0, 1, 3)).reshape(s0, s1, dim)


# speedup vs baseline: 2.7924x; 2.7924x over previous
import functools

import jax
import jax.numpy as jnp
from jax import lax
from jax.experimental import pallas as pl
from jax.experimental.pallas import tpu as pltpu
from jax.experimental.pallas import tpu_sc as plsc

_NUM_CORES = 2
_NUM_SUBCORES = 16
_NUM_WORKERS = _NUM_CORES * _NUM_SUBCORES
_PAD = 128

_GRP = 4
_NBUF = 2


def _make_lookup(num_rows: int, dim: int, s0: int, s1: int):
  assert s0 % _NUM_WORKERS == 0
  rows_per_w = s0 // _NUM_WORKERS
  assert rows_per_w % (_GRP * _NBUF) == 0
  n_groups = rows_per_w // (_GRP * _NBUF)
  assert s1 % 8 == 0

  mesh = plsc.VectorSubcoreMesh(
      core_axis_name="c", subcore_axis_name="s", num_cores=_NUM_CORES)

  @functools.partial(
      pl.kernel,
      mesh=mesh,
      compiler_params=pltpu.CompilerParams(use_tc_tiling_on_sc=False),
      out_type=jax.ShapeDtypeStruct((s0, s1, _PAD), jnp.float32),
      scratch_types=[
          pltpu.VMEM((rows_per_w, s1), jnp.int32),
          pltpu.VMEM((_NBUF, _GRP, s1, dim), jnp.float32),
          pltpu.SemaphoreType.DMA,
          pltpu.SemaphoreType.DMA,
      ],
  )
  def lookup(table_hbm, idx_hbm, out_hbm, idx_v, rows_v, gsem0, gsem1):
    gsems = (gsem0, gsem1)
    wid = lax.axis_index("s") * _NUM_CORES + lax.axis_index("c")
    wbase = wid * rows_per_w
    pltpu.sync_copy(idx_hbm.at[pl.ds(wbase, rows_per_w)], idx_v)

    @pl.loop(0, n_groups)
    def _group(g):
      descs = [[] for _ in range(_NBUF)]
      for b in range(_NBUF):
        for j in range(_GRP):
          row = (g * _NBUF + b) * _GRP + j
          descs[b].append(
              pltpu.async_copy(
                  table_hbm.at[idx_v.at[row]],
                  rows_v.at[b].at[j], gsems[b]))
      for b in range(_NBUF):
        for d in descs[b]:
          d.wait()
        row0 = (g * _NBUF + b) * _GRP
        pltpu.sync_copy(
            rows_v.at[b],
            out_hbm.at[pl.ds(wbase + row0, _GRP), :, pl.ds(0, dim)])

  return lookup


def kernel(token_ids, embedding_matrix):
  s0, s1 = token_ids.shape
  num_rows, dim = embedding_matrix.shape
  idx2 = token_ids.astype(jnp.int32) * 2
  proj = jnp.eye(dim, _PAD, dtype=jnp.float32)
  tbl = jax.lax.dot(embedding_matrix, proj,
                    precision=jax.lax.Precision.HIGH)
  tbl2 = tbl.reshape(num_rows * 2, dim)
  lookup = _make_lookup(num_rows, dim, s0, s1)
  padded = lookup(tbl2, idx2)
  return padded[:, :, :dim]
